# baseline (speedup 1.0000x reference)
import functools

import jax
import jax.numpy as jnp
from jax import lax
from jax.experimental import pallas as pl
from jax.experimental.pallas import tpu as pltpu
from jax.experimental.pallas import tpu_sc as plsc

_EPS = 1e-5


@functools.cache
def _make_sc_gather_pair(B, D, V):
    info = plsc.get_sparse_core_info()
    NC, NS, L = info.num_cores, info.num_subcores, info.num_lanes
    NW = NC * NS
    FPW = 2 * D // NW
    WPT = D // FPW
    QTR = B // 4
    assert B % (16 * L) == 0 and 2 * D % NW == 0

    mesh = plsc.VectorSubcoreMesh(core_axis_name="c", subcore_axis_name="s")

    @functools.partial(
        pl.kernel,
        mesh=mesh,
        compiler_params=pltpu.CompilerParams(needs_layout_passes=False),
        out_type=(
            jax.ShapeDtypeStruct((D, B), jnp.float32),
            jax.ShapeDtypeStruct((D, B), jnp.float32),
        ),
        scratch_types=[
            pltpu.VMEM((V,), jnp.float32),
            pltpu.VMEM((B,), jnp.int32),
            pltpu.VMEM((2, QTR), jnp.float32),
            pltpu.SemaphoreType.DMA,
        ],
    )
    def sc_gather(uid, iid, tu, ti, ou, oi, colbuf, idbuf, outbuf, osem):
        wid = lax.axis_index("s") * NC + lax.axis_index("c")
        tbl = wid // WPT
        d0 = (wid % WPT) * FPW
        for t, (tref, idsrc, oref) in enumerate(((tu, uid, ou),
                                                 (ti, iid, oi))):
            @pl.when(tbl == t)
            def _():
                pltpu.sync_copy(idsrc, idbuf)

                def wait_one(h):
                    pltpu.make_async_copy(
                        oref.at[0, pl.ds(0, QTR)], outbuf.at[h % 2],
                        osem).wait()

                def per_feature(f, carry):
                    d = d0 + f
                    pltpu.sync_copy(tref.at[d], colbuf)
                    for h in range(4):
                        if h >= 2:
                            wait_one(h)
                        else:
                            @pl.when(f > 0)
                            def _():
                                wait_one(h)

                        @plsc.parallel_loop(0, QTR, step=4 * L, unroll=4)
                        def _(i):
                            for j in range(4):
                                idx = idbuf[pl.ds(h * QTR + i + j * L, L)]
                                outbuf[h % 2, pl.ds(i + j * L, L)] = (
                                    plsc.load_gather(colbuf, [idx]))
                        pltpu.async_copy(
                            outbuf.at[h % 2],
                            oref.at[d, pl.ds(h * QTR, QTR)], osem)
                    return carry

                lax.fori_loop(0, FPW, per_feature, 0)
                wait_one(0)
                wait_one(1)

    return sc_gather


def _tower_body(muT, miT, w0a, w0b, b0, w1, b1, w2, b2, out):
    f32 = jnp.float32
    h = jnp.dot(w0a[...], muT[...], preferred_element_type=f32)
    h = h + jnp.dot(w0b[...], miT[...], preferred_element_type=f32)
    h = jnp.maximum(h + b0[...], 0.0)
    h = jnp.maximum(
        jnp.dot(w1[...], h, preferred_element_type=f32) + b1[...], 0.0)
    out[...] = jnp.maximum(
        jnp.dot(w2[...], h, preferred_element_type=f32) + b2[...], 0.0)


def _logit_body(guT, giT, hT, wg, wm, bo, out):
    f32 = jnp.float32
    g = guT[...] * giT[...]
    logit = (jnp.dot(wg[...], g, preferred_element_type=f32)
             + jnp.dot(wm[...], hT[...], preferred_element_type=f32)
             + bo[...])
    out[...] = 1.0 / (1.0 + jnp.exp(-logit))


def _tc_tower(muT, miT, w0a, w0b, b0, w1, b1, w2, b2):
    D, B = muT.shape
    BLK = 2048
    H0, H1, H2 = w0a.shape[0], w1.shape[0], w2.shape[0]
    full = lambda s: pl.BlockSpec(s, lambda i: (0, 0))
    return pl.pallas_call(
        _tower_body,
        grid=(B // BLK,),
        in_specs=[
            pl.BlockSpec((D, BLK), lambda i: (0, i)),
            pl.BlockSpec((D, BLK), lambda i: (0, i)),
            full((H0, D)), full((H0, D)), full((H0, 1)),
            full((H1, H0)), full((H1, 1)),
            full((H2, H1)), full((H2, 1)),
        ],
        out_specs=pl.BlockSpec((H2, BLK), lambda i: (0, i)),
        out_shape=jax.ShapeDtypeStruct((H2, B), jnp.float32),
    )(muT, miT, w0a, w0b, b0, w1, b1, w2, b2)


def _tc_logit(guT, giT, hT, wg, wm, bo2):
    D, B = guT.shape
    BLK = 2048
    H2 = hT.shape[0]
    full = lambda s: pl.BlockSpec(s, lambda i: (0, 0))
    out = pl.pallas_call(
        _logit_body,
        grid=(B // BLK,),
        in_specs=[
            pl.BlockSpec((D, BLK), lambda i: (0, i)),
            pl.BlockSpec((D, BLK), lambda i: (0, i)),
            pl.BlockSpec((H2, BLK), lambda i: (0, i)),
            full((1, D)), full((1, H2)), full((1, 1)),
        ],
        out_specs=pl.BlockSpec((1, BLK), lambda i: (0, i)),
        out_shape=jax.ShapeDtypeStruct((1, B), jnp.float32),
    )(guT, giT, hT, wg, wm, bo2)
    return out.reshape(B)


def kernel(user_ids, item_ids, gmf_user_w, gmf_item_w, mlp_user_w, mlp_item_w,
           W0, b0, g0, be0, W1, b1, g1, be1, W2, b2, g2, be2, Wo, bo):
    B = user_ids.shape[0]
    V, D = gmf_user_w.shape
    uid = user_ids.astype(jnp.int32)
    iid = item_ids.astype(jnp.int32)

    t0 = jnp.swapaxes(gmf_user_w, 0, 1)
    t1 = jnp.swapaxes(gmf_item_w, 0, 1)
    t2 = jnp.swapaxes(mlp_user_w, 0, 1)
    t3 = jnp.swapaxes(mlp_item_w, 0, 1)

    gather = _make_sc_gather_pair(B, D, V)
    muT, miT = gather(uid, iid, t2, t3)
    guT, giT = gather(uid, iid, t0, t1)

    inv = 1.0 / jnp.sqrt(jnp.float32(1.0 + _EPS))
    def fold(W, b, g, be):
        s = g * inv
        return W * s[:, None], (b * s + be)[:, None]
    w0f, b0c = fold(W0, b0, g0, be0)
    w1f, b1c = fold(W1, b1, g1, be1)
    w2f, b2c = fold(W2, b2, g2, be2)
    w0a, w0b = w0f[:, :D], w0f[:, D:]
    wg = Wo[:, :D]
    wm = Wo[:, D:]
    bo2 = bo[None, :]

    hT = _tc_tower(muT, miT, w0a, w0b, b0c, w1f, b1c, w2f, b2c)
    return _tc_logit(guT, giT, hT, wg, wm, bo2)
